# Initial kernel scaffold; baseline (speedup 1.0000x reference)
import jax
import jax.numpy as jnp
from jax.experimental import pallas as pl


def kernel(tokens, embodiment_ids, degree_counts_by_id, embedding):
    raise NotImplementedError("write your pallas kernel here")


# speedup vs baseline: 2.2404x; 2.2404x over previous
import functools

import jax
import jax.numpy as jnp
from jax import lax
from jax.experimental import pallas as pl
from jax.experimental.pallas import tpu as pltpu

_BM = 2048


def _body(idsr_ref, dctr_ref, emb_ref, tok_ref, out_ref):
    bm = tok_ref.shape[0]
    n_emb = dctr_ref.shape[1]
    n_rows = emb_ref.shape[0]
    ids_rep = idsr_ref[...]
    e_io = lax.broadcasted_iota(jnp.int32, (bm, n_emb), 1)
    sel = (ids_rep == e_io).astype(jnp.int32)
    idx = jnp.sum(dctr_ref[...] * sel, axis=1, keepdims=True)
    k_io = lax.broadcasted_iota(jnp.int32, (bm, n_rows), 1)
    oh = (idx == k_io).astype(jnp.bfloat16)
    pe = jnp.dot(oh, emb_ref[...], preferred_element_type=jnp.float32)
    out_ref[...] = tok_ref[...] + pe


def kernel(tokens, embodiment_ids, degree_counts_by_id, embedding):
    seq_len, batch, d_model = tokens.shape
    n_emb = degree_counts_by_id.shape[0]
    n_rows = embedding.shape[0]
    m = seq_len * batch

    tok2 = tokens.reshape(m, d_model)
    ids_rep = jnp.tile(embodiment_ids[:, None], (seq_len, 1))
    dct_rep = jnp.repeat(degree_counts_by_id.T, batch, axis=0)
    emb_bf = embedding.astype(jnp.bfloat16)

    grid = (m // _BM,)
    out2 = pl.pallas_call(
        _body,
        grid=grid,
        in_specs=[
            pl.BlockSpec((_BM, 1), lambda i: (i, 0)),
            pl.BlockSpec((_BM, n_emb), lambda i: (i, 0)),
            pl.BlockSpec((n_rows, d_model), lambda i: (0, 0)),
            pl.BlockSpec((_BM, d_model), lambda i: (i, 0)),
        ],
        out_specs=pl.BlockSpec((_BM, d_model), lambda i: (i, 0)),
        out_shape=jax.ShapeDtypeStruct((m, d_model), jnp.float32),
    )(ids_rep, dct_rep, emb_bf, tok2)
    return out2.reshape(seq_len, batch, d_model)


# speedup vs baseline: 3.0769x; 1.3734x over previous
import functools

import jax
import jax.numpy as jnp
from jax import lax
from jax.experimental import pallas as pl
from jax.experimental.pallas import tpu as pltpu

_BM = 2048


def _body(ids_ref, dct_ref, emb_ref, tok_ref, out_ref):
    bm = tok_ref.shape[0]
    nb = ids_ref.shape[0]
    bs = dct_ref.shape[0]
    n_emb = dct_ref.shape[1]
    n_rows = emb_ref.shape[0]

    r_io = lax.broadcasted_iota(jnp.int32, (bm, nb), 0)
    b_io = lax.broadcasted_iota(jnp.int32, (bm, nb), 1)
    rep_b = (r_io % nb == b_io).astype(jnp.float32)
    ids_rep = jnp.dot(rep_b, ids_ref[...], preferred_element_type=jnp.float32)

    rs_io = lax.broadcasted_iota(jnp.int32, (bm, bs), 0)
    s_io = lax.broadcasted_iota(jnp.int32, (bm, bs), 1)
    rep_s = (rs_io // nb == s_io).astype(jnp.float32)
    dc_rows = jnp.dot(rep_s, dct_ref[...], preferred_element_type=jnp.float32)

    e_io = lax.broadcasted_iota(jnp.int32, (bm, n_emb), 1).astype(jnp.float32)
    sel = (ids_rep == e_io).astype(jnp.float32)
    idx = jnp.sum(dc_rows * sel, axis=1, keepdims=True)

    k_io = lax.broadcasted_iota(jnp.int32, (bm, n_rows), 1).astype(jnp.float32)
    oh = (idx == k_io).astype(jnp.bfloat16)
    pe = jnp.dot(oh, emb_ref[...], preferred_element_type=jnp.float32)
    out_ref[...] = tok_ref[...] + pe


def kernel(tokens, embodiment_ids, degree_counts_by_id, embedding):
    seq_len, batch, d_model = tokens.shape
    n_emb = degree_counts_by_id.shape[0]
    n_rows = embedding.shape[0]
    m = seq_len * batch
    bs = _BM // batch

    tok2 = tokens.reshape(m, d_model)
    ids_f = embodiment_ids.astype(jnp.float32)[:, None]
    dct_f = degree_counts_by_id.T.astype(jnp.float32)
    emb_bf = embedding.astype(jnp.bfloat16)

    grid = (m // _BM,)
    out2 = pl.pallas_call(
        _body,
        grid=grid,
        in_specs=[
            pl.BlockSpec((batch, 1), lambda i: (0, 0)),
            pl.BlockSpec((bs, n_emb), lambda i: (i, 0)),
            pl.BlockSpec((n_rows, d_model), lambda i: (0, 0)),
            pl.BlockSpec((_BM, d_model), lambda i: (i, 0)),
        ],
        out_specs=pl.BlockSpec((_BM, d_model), lambda i: (i, 0)),
        out_shape=jax.ShapeDtypeStruct((m, d_model), jnp.float32),
    )(ids_f, dct_f, emb_bf, tok2)
    return out2.reshape(seq_len, batch, d_model)
